# Initial kernel scaffold; baseline (speedup 1.0000x reference)
import jax
import jax.numpy as jnp
from jax.experimental import pallas as pl


def kernel(inputs, table):
    raise NotImplementedError("write your pallas kernel here")


# speedup vs baseline: 1.4331x; 1.4331x over previous
import functools

import jax
import jax.numpy as jnp
from jax import lax
from jax.experimental import pallas as pl
from jax.experimental.pallas import tpu as pltpu
from jax.experimental.pallas import tpu_sc as plsc

_INPUT_DIM = 1000000
_EMBED_DIM = 32
_LANES = 16

_info = plsc.get_sparse_core_info()
_NC, _NS = _info.num_cores, _info.num_subcores
_NW = _NC * _NS

_CHUNK = 128


def _hash16(x):
    h = x * jnp.int32(-1640531535)
    hf = h.astype(jnp.float32)
    hf = jnp.where(h < 0, hf + jnp.float32(4294967296.0), hf)
    q = (hf * jnp.float32(1e-6)).astype(jnp.int32)
    r = h - q * jnp.int32(_INPUT_DIM)
    r = jnp.where(r < 0, r + jnp.int32(_INPUT_DIM), r)
    r = jnp.where(r >= jnp.int32(_INPUT_DIM), r - jnp.int32(_INPUT_DIM), r)
    return r


def _make_kernel(batch_flat):
    bpw = batch_flat // _NW
    nchunk = bpw // _CHUNK
    mesh = plsc.VectorSubcoreMesh(core_axis_name="c", subcore_axis_name="s")

    @functools.partial(
        pl.kernel,
        mesh=mesh,
        compiler_params=pltpu.CompilerParams(use_tc_tiling_on_sc=False),
        out_type=jax.ShapeDtypeStruct((batch_flat, _EMBED_DIM), jnp.float32),
        scratch_types=[
            pltpu.VMEM((nchunk, _CHUNK), jnp.int32),
            pltpu.VMEM((_CHUNK, _EMBED_DIM), jnp.float32),
            pltpu.SemaphoreType.DMA,
        ],
    )
    def k(ids_hbm, table_hbm, out_hbm, idx_v, rows_v, sem):
        wid = lax.axis_index("s") * _NC + lax.axis_index("c")
        base = wid * bpw
        pltpu.sync_copy(ids_hbm.at[wid], idx_v)

        def hash_step(j, carry):
            for i in range(_CHUNK // _LANES):
                s = pl.ds(i * _LANES, _LANES)
                idx_v[j, s] = _hash16(idx_v[j, s])
            return carry

        lax.fori_loop(0, nchunk, hash_step, 0)

        def gather_step(j, carry):
            pltpu.async_copy(table_hbm.at[idx_v.at[j]], rows_v, sem).wait()
            pltpu.sync_copy(rows_v, out_hbm.at[pl.ds(base + j * _CHUNK, _CHUNK)])
            return carry

        lax.fori_loop(0, nchunk, gather_step, 0)

    return k


def kernel(inputs, table):
    b, f = inputs.shape
    bf = b * f
    ids = inputs.reshape(_NW, bf // (_NW * _CHUNK), _CHUNK)
    out = _make_kernel(bf)(ids, table)
    return out.reshape(b, f, _EMBED_DIM)


# speedup vs baseline: 1.5734x; 1.0979x over previous
import functools

import jax
import jax.numpy as jnp
from jax import lax
from jax.experimental import pallas as pl
from jax.experimental.pallas import tpu as pltpu
from jax.experimental.pallas import tpu_sc as plsc

_INPUT_DIM = 1000000
_EMBED_DIM = 32
_LANES = 16

_info = plsc.get_sparse_core_info()
_NC, _NS = _info.num_cores, _info.num_subcores
_NW = _NC * _NS

_CHUNK = 128


def _hash16(x):
    h = x * jnp.int32(-1640531535)
    hf = h.astype(jnp.float32)
    hf = jnp.where(h < 0, hf + jnp.float32(4294967296.0), hf)
    q = (hf * jnp.float32(1e-6)).astype(jnp.int32)
    r = h - q * jnp.int32(_INPUT_DIM)
    r = jnp.where(r < 0, r + jnp.int32(_INPUT_DIM), r)
    r = jnp.where(r >= jnp.int32(_INPUT_DIM), r - jnp.int32(_INPUT_DIM), r)
    return r


_NBUF = 8


def _make_kernel(batch_flat):
    bpw = batch_flat // _NW
    nchunk = bpw // _CHUNK
    ngroups = nchunk // _NBUF
    mesh = plsc.VectorSubcoreMesh(core_axis_name="c", subcore_axis_name="s")

    @functools.partial(
        pl.kernel,
        mesh=mesh,
        compiler_params=pltpu.CompilerParams(use_tc_tiling_on_sc=False),
        out_type=jax.ShapeDtypeStruct((batch_flat, _EMBED_DIM), jnp.float32),
        scratch_types=[
            pltpu.VMEM((nchunk, _CHUNK), jnp.int32),
            pltpu.VMEM((_NBUF, _CHUNK, _EMBED_DIM), jnp.float32),
        ]
        + [pltpu.SemaphoreType.DMA] * _NBUF,
    )
    def k(ids_hbm, table_hbm, out_hbm, idx_v, rows_v, *sems):
        wid = lax.axis_index("s") * _NC + lax.axis_index("c")
        base = wid * bpw
        pltpu.sync_copy(ids_hbm.at[wid], idx_v)

        def hash_step(j, carry):
            for i in range(_CHUNK // _LANES):
                s = pl.ds(i * _LANES, _LANES)
                idx_v[j, s] = _hash16(idx_v[j, s])
            return carry

        lax.fori_loop(0, nchunk, hash_step, 0)

        def start(j, b):
            pltpu.async_copy(table_hbm.at[idx_v.at[j]], rows_v.at[b], sems[b])

        def finish(j, b):
            pltpu.make_async_copy(
                table_hbm.at[idx_v.at[j]], rows_v.at[b], sems[b]
            ).wait()
            pltpu.sync_copy(rows_v.at[b], out_hbm.at[pl.ds(base + j * _CHUNK, _CHUNK)])

        for b in range(_NBUF):
            start(b, b)

        def group_step(g, carry):
            for b in range(_NBUF):
                j = g * _NBUF + b
                finish(j, b)
                start(j + _NBUF, b)
            return carry

        lax.fori_loop(0, ngroups - 1, group_step, 0)

        for b in range(_NBUF):
            finish((ngroups - 1) * _NBUF + b, b)

    return k


def kernel(inputs, table):
    b, f = inputs.shape
    bf = b * f
    ids = inputs.reshape(_NW, bf // (_NW * _CHUNK), _CHUNK)
    out = _make_kernel(bf)(ids, table)
    return out.reshape(b, f, _EMBED_DIM)
